# Initial kernel scaffold; baseline (speedup 1.0000x reference)
import jax
import jax.numpy as jnp
from jax.experimental import pallas as pl


def kernel(x, proj_W, gate_W, gate_b, path_W1, path_b1, path_W2, path_b2, threshold):
    raise NotImplementedError("write your pallas kernel here")


# speedup vs baseline: 13.5063x; 13.5063x over previous
import numpy as np
import jax
import jax.numpy as jnp
from jax.experimental import pallas as pl
from jax.experimental.pallas import tpu as pltpu

B = 32; N_CH = 22; N_P = 20; N = N_CH * N_P; D = 128; H = 4; HD = D // H; F = 3
ALPHA = 0.15; TOP_P = 0.85; DIST_THRESH = 0.55
K = max(1, int(N * ALPHA))
T_BS = 24

_TCP_PAIRS = [('FP1','F7'),('F7','T3'),('T3','T5'),('T5','O1'),('FP2','F8'),('F8','T4'),('T4','T6'),('T6','O2'),('FP1','F3'),('F3','C3'),('C3','P3'),('P3','O1'),('FP2','F4'),('F4','C4'),('C4','P4'),('P4','O2'),('A1','T3'),('T3','C3'),('C3','CZ'),('CZ','C4'),('C4','T4'),('T4','A2')]
_E3D = {'FP1':(-0.31,0.95,0.0),'FP2':(0.31,0.95,0.0),'F7':(-0.81,0.59,0.0),'F3':(-0.55,0.67,0.5),'FZ':(0.0,0.71,0.71),'F4':(0.55,0.67,0.5),'F8':(0.81,0.59,0.0),'T3':(-1.0,0.0,0.0),'C3':(-0.57,0.0,0.82),'CZ':(0.0,0.0,1.0),'C4':(0.57,0.0,0.82),'T4':(1.0,0.0,0.0),'T5':(-0.81,-0.59,0.0),'P3':(-0.55,-0.67,0.5),'PZ':(0.0,-0.71,0.71),'P4':(0.55,-0.67,0.5),'T6':(0.81,-0.59,0.0),'O1':(-0.31,-0.95,0.0),'O2':(0.31,-0.95,0.0),'A1':(-1.05,0.0,-0.3),'A2':(1.05,0.0,-0.3)}


def _build_static_masks():
    ch_pos = np.array([(np.array(_E3D[a]) + np.array(_E3D[b])) / 2.0 for a, b in _TCP_PAIRS])
    dif = ch_pos[:, None, :] - ch_pos[None, :, :]
    ch_dist = np.sqrt((dif ** 2).sum(-1))
    ch_adj = (ch_dist < DIST_THRESH).astype(np.float32)
    np.fill_diagonal(ch_adj, 0.0)
    smask = np.zeros((N, N), np.float32)
    for t in range(N_P):
        idx = np.arange(N_CH) * N_P + t
        smask[np.ix_(idx, idx)] = ch_adj
    tmask = np.zeros((N, N), np.float32)
    for ch in range(N_CH):
        for p in range(N_P):
            node = ch * N_P + p
            if p > 0:
                tmask[node, node - 1] = 1.0
            if p < N_P - 1:
                tmask[node, node + 1] = 1.0
    return jnp.asarray(smask), jnp.asarray(tmask)


_SMASK, _TMASK = _build_static_masks()

_HI = jax.lax.Precision.HIGHEST


def _erf(v):
    s = jnp.where(v >= 0.0, 1.0, -1.0)
    a = jnp.abs(v)
    t = 1.0 / (1.0 + 0.3275911 * a)
    poly = t * (0.254829592 + t * (-0.284496736 + t * (1.421413741
               + t * (-1.453152027 + t * 1.061405429))))
    return s * (1.0 - poly * jnp.exp(-a * a))


def _gelu_exact(v):
    return 0.5 * v * (1.0 + _erf(v * np.float32(1.0 / np.sqrt(2.0))))


def _eye():
    r = jax.lax.broadcasted_iota(jnp.int32, (N, N), 0)
    c = jax.lax.broadcasted_iota(jnp.int32, (N, N), 1)
    return jnp.where(r == c, 1.0, 0.0).astype(jnp.float32)


def _col2row(v, eye):
    return jax.lax.dot_general(v, eye, (((0,), (0,)), ((), ())), precision=_HI)


def _head_d2(xb, pw):
    z = jax.lax.dot(xb, pw)
    sq = jnp.sum(z * z, axis=-1, keepdims=True)
    g = jax.lax.dot_general(z, z, (((1,), (1,)), ((), ())))
    return z, sq, g


def _sum_body(x_ref, pw_ref, sums_ref):
    xb = x_ref[0]
    eye = _eye()
    parts = []
    for h in range(H):
        _, sq, g = _head_d2(xb, pw_ref[h])
        sq_t = _col2row(sq, eye)
        d2 = sq + sq_t - 2.0 * g
        dist = jnp.sqrt(jnp.maximum(d2, 1e-12))
        parts.append(jnp.sum(dist).reshape(1, 1))
    row = jnp.concatenate(parts + [jnp.zeros((1, 8 - H), jnp.float32)], axis=1)
    sums_ref[0] = row


def _main_body(x_ref, pw_ref, gw_ref, gb_ref, pw1_ref, pb1_ref, pw2_ref,
               pb2_ref, thr_ref, sums_ref, smask_ref, tmask_ref, out_ref):
    xb = x_ref[0]
    eye = _eye()
    ones_col = jnp.ones((N, 1), jnp.float32)
    head_sums = jnp.sum(sums_ref[...], axis=0)

    adj = None
    for h in range(H):
        _, sq, g = _head_d2(xb, pw_ref[h])
        sq_t = _col2row(sq, eye)
        d2 = sq + sq_t - 2.0 * g
        lo = jnp.zeros((N, 1), jnp.float32)
        hi = jnp.max(d2, axis=1, keepdims=True)
        for _ in range(T_BS):
            mid = 0.5 * (lo + hi)
            cmpf = jnp.where(d2 <= mid, 1.0, 0.0)
            cnt = jax.lax.dot(cmpf, ones_col)
            pred = cnt >= K
            lo = jnp.where(pred, lo, mid)
            hi = jnp.where(pred, mid, hi)
        t = hi
        m_asym = jnp.where(d2 <= t, 1.0, 0.0)
        m_t = jax.lax.dot_general(m_asym, eye, (((0,), (0,)), ((), ())))
        maskf = jnp.maximum(m_asym, m_t)
        dist = jnp.sqrt(jnp.maximum(d2, 1e-12))
        mean_h = head_sums[0, h] / float(B * N * N)
        sim = jnp.exp(-(dist * dist) / (2.0 * mean_h * mean_h + 1e-08))
        contrib = sim * maskf
        adj = contrib if adj is None else adj + contrib
    adj = adj / float(H)

    logits = jax.lax.dot(xb, gw_ref[...]) + gb_ref[...]
    probs = jax.nn.softmax(logits, axis=-1)
    p0 = probs[:, 0:1]; p1 = probs[:, 1:2]; p2 = probs[:, 2:3]
    z0 = jnp.zeros_like(p0)
    bef0 = jnp.where(p1 > p0, p1, z0) + jnp.where(p2 > p0, p2, z0)
    bef1 = jnp.where(p0 >= p1, p0, z0) + jnp.where(p2 > p1, p2, z0)
    bef2 = jnp.where(p0 >= p2, p0, z0) + jnp.where(p1 >= p2, p1, z0)
    w0 = jnp.where(bef0 < TOP_P, p0, z0)
    w1 = jnp.where(bef1 < TOP_P, p1, z0)
    w2 = jnp.where(bef2 < TOP_P, p2, z0)
    denom = w0 + w1 + w2 + 1e-08
    w0 = w0 / denom; w1 = w1 / denom; w2 = w2 / denom

    h1 = _gelu_exact(jax.lax.dot(xb, pw1_ref[...]) + pb1_ref[...])
    sc = jax.nn.sigmoid(jax.lax.dot(h1, pw2_ref[...]) + pb2_ref[0, 0])
    sc_t = _col2row(sc, eye)
    gate = jax.nn.sigmoid((sc * sc_t - thr_ref[0, 0]) * 10.0)

    out_ref[0] = ((adj * gate) * w0 + (adj * smask_ref[...]) * w1
                  + (adj * tmask_ref[...]) * w2)


def _full(shape):
    nd = len(shape)
    return pl.BlockSpec(shape, lambda b, _n=nd: (0,) * _n)


def kernel(x, proj_W, gate_W, gate_b, path_W1, path_b1, path_W2, path_b2, threshold):
    gate_b2 = gate_b.reshape(1, F)
    path_b1_2 = path_b1.reshape(1, D // 2)
    path_b2_2 = path_b2.reshape(1, 1)
    thr2 = threshold.reshape(1, 1)

    sums = pl.pallas_call(
        _sum_body,
        grid=(B,),
        in_specs=[pl.BlockSpec((1, N, D), lambda b: (b, 0, 0)), _full((H, D, HD))],
        out_specs=pl.BlockSpec((1, 1, 8), lambda b: (b, 0, 0)),
        out_shape=jax.ShapeDtypeStruct((B, 1, 8), jnp.float32),
        compiler_params=pltpu.CompilerParams(
            dimension_semantics=("parallel",)),
    )(x, proj_W)

    out = pl.pallas_call(
        _main_body,
        grid=(B,),
        in_specs=[
            pl.BlockSpec((1, N, D), lambda b: (b, 0, 0)),
            _full((H, D, HD)),
            _full((D, F)),
            _full((1, F)),
            _full((D, D // 2)),
            _full((1, D // 2)),
            _full((D // 2, 1)),
            _full((1, 1)),
            _full((1, 1)),
            _full((B, 1, 8)),
            _full((N, N)),
            _full((N, N)),
        ],
        out_specs=pl.BlockSpec((1, N, N), lambda b: (b, 0, 0)),
        out_shape=jax.ShapeDtypeStruct((B, N, N), jnp.float32),
        compiler_params=pltpu.CompilerParams(
            dimension_semantics=("parallel",)),
    )(x, proj_W, gate_W, gate_b2, path_W1, path_b1_2, path_W2, path_b2_2,
      thr2, sums, _SMASK, _TMASK)
    return out


# speedup vs baseline: 25.5535x; 1.8920x over previous
import numpy as np
import jax
import jax.numpy as jnp
from jax.experimental import pallas as pl
from jax.experimental.pallas import tpu as pltpu

B = 32; N_CH = 22; N_P = 20; N = N_CH * N_P; D = 128; H = 4; HD = D // H; F = 3
ALPHA = 0.15; TOP_P = 0.85; DIST_THRESH = 0.55
K = max(1, int(N * ALPHA))
T_BS = 22

_TCP_PAIRS = [('FP1','F7'),('F7','T3'),('T3','T5'),('T5','O1'),('FP2','F8'),('F8','T4'),('T4','T6'),('T6','O2'),('FP1','F3'),('F3','C3'),('C3','P3'),('P3','O1'),('FP2','F4'),('F4','C4'),('C4','P4'),('P4','O2'),('A1','T3'),('T3','C3'),('C3','CZ'),('CZ','C4'),('C4','T4'),('T4','A2')]
_E3D = {'FP1':(-0.31,0.95,0.0),'FP2':(0.31,0.95,0.0),'F7':(-0.81,0.59,0.0),'F3':(-0.55,0.67,0.5),'FZ':(0.0,0.71,0.71),'F4':(0.55,0.67,0.5),'F8':(0.81,0.59,0.0),'T3':(-1.0,0.0,0.0),'C3':(-0.57,0.0,0.82),'CZ':(0.0,0.0,1.0),'C4':(0.57,0.0,0.82),'T4':(1.0,0.0,0.0),'T5':(-0.81,-0.59,0.0),'P3':(-0.55,-0.67,0.5),'PZ':(0.0,-0.71,0.71),'P4':(0.55,-0.67,0.5),'T6':(0.81,-0.59,0.0),'O1':(-0.31,-0.95,0.0),'O2':(0.31,-0.95,0.0),'A1':(-1.05,0.0,-0.3),'A2':(1.05,0.0,-0.3)}


def _build_static_masks():
    ch_pos = np.array([(np.array(_E3D[a]) + np.array(_E3D[b])) / 2.0 for a, b in _TCP_PAIRS])
    dif = ch_pos[:, None, :] - ch_pos[None, :, :]
    ch_dist = np.sqrt((dif ** 2).sum(-1))
    ch_adj = (ch_dist < DIST_THRESH).astype(np.float32)
    np.fill_diagonal(ch_adj, 0.0)
    smask = np.zeros((N, N), np.float32)
    for t in range(N_P):
        idx = np.arange(N_CH) * N_P + t
        smask[np.ix_(idx, idx)] = ch_adj
    tmask = np.zeros((N, N), np.float32)
    for ch in range(N_CH):
        for p in range(N_P):
            node = ch * N_P + p
            if p > 0:
                tmask[node, node - 1] = 1.0
            if p < N_P - 1:
                tmask[node, node + 1] = 1.0
    return jnp.asarray(smask), jnp.asarray(tmask)


_SMASK, _TMASK = _build_static_masks()


def _erf(v):
    s = jnp.where(v >= 0.0, 1.0, -1.0)
    a = jnp.abs(v)
    t = 1.0 / (1.0 + 0.3275911 * a)
    poly = t * (0.254829592 + t * (-0.284496736 + t * (1.421413741
               + t * (-1.453152027 + t * 1.061405429))))
    return s * (1.0 - poly * jnp.exp(-a * a))


def _gelu_exact(v):
    return 0.5 * v * (1.0 + _erf(v * np.float32(1.0 / np.sqrt(2.0))))


def _head_d2(xb, pw):
    z = jax.lax.dot(xb, pw)
    sq = jnp.sum(z * z, axis=-1, keepdims=True)
    g = jax.lax.dot_general(z, z, (((1,), (1,)), ((), ())))
    return z, sq, g


def _sum_body(x_ref, pw_ref, sums_ref):
    xb = x_ref[0]
    parts = []
    for h in range(H):
        _, sq, g = _head_d2(xb, pw_ref[h])
        sq_t = jnp.transpose(sq)
        d2 = sq + sq_t - 2.0 * g
        dist = jnp.sqrt(jnp.maximum(d2, 1e-12))
        parts.append(jnp.sum(dist).reshape(1, 1))
    row = jnp.concatenate(parts + [jnp.zeros((1, 8 - H), jnp.float32)], axis=1)
    sums_ref[0] = row


def _main_body(x_ref, pw_ref, gw_ref, gb_ref, pw1_ref, pb1_ref, pw2_ref,
               pb2_ref, thr_ref, sums_ref, smask_ref, tmask_ref, out_ref):
    xb = x_ref[0]
    head_sums = jnp.sum(sums_ref[...], axis=0)

    d2s = []
    for h in range(H):
        _, sq, g = _head_d2(xb, pw_ref[h])
        sq_t = jnp.transpose(sq)
        d2s.append(sq + sq_t - 2.0 * g)
    d2a = jnp.concatenate(d2s, axis=0)

    ones_col = jnp.ones((N, 1), jnp.float32)
    lo = jnp.zeros((H * N, 1), jnp.float32)
    hi = jnp.max(d2a, axis=1, keepdims=True)
    for _ in range(T_BS):
        mid = 0.5 * (lo + hi)
        cmpf = jnp.where(d2a <= mid, 1.0, 0.0)
        cnt = jax.lax.dot(cmpf, ones_col)
        pred = cnt >= K
        lo = jnp.where(pred, lo, mid)
        hi = jnp.where(pred, mid, hi)

    adj = None
    for h in range(H):
        d2 = d2s[h]
        t = jax.lax.slice(hi, (h * N, 0), ((h + 1) * N, 1))
        t_t = jnp.transpose(t)
        maskf = jnp.where(d2 <= jnp.maximum(t, t_t), 1.0, 0.0)
        dist = jnp.sqrt(jnp.maximum(d2, 1e-12))
        mean_h = head_sums[0, h] / float(B * N * N)
        sim = jnp.exp(-(dist * dist) / (2.0 * mean_h * mean_h + 1e-08))
        contrib = sim * maskf
        adj = contrib if adj is None else adj + contrib
    adj = adj / float(H)

    logits = jax.lax.dot(xb, gw_ref[...]) + gb_ref[...]
    probs = jax.nn.softmax(logits, axis=-1)
    p0 = probs[:, 0:1]; p1 = probs[:, 1:2]; p2 = probs[:, 2:3]
    z0 = jnp.zeros_like(p0)
    bef0 = jnp.where(p1 > p0, p1, z0) + jnp.where(p2 > p0, p2, z0)
    bef1 = jnp.where(p0 >= p1, p0, z0) + jnp.where(p2 > p1, p2, z0)
    bef2 = jnp.where(p0 >= p2, p0, z0) + jnp.where(p1 >= p2, p1, z0)
    w0 = jnp.where(bef0 < TOP_P, p0, z0)
    w1 = jnp.where(bef1 < TOP_P, p1, z0)
    w2 = jnp.where(bef2 < TOP_P, p2, z0)
    denom = w0 + w1 + w2 + 1e-08
    w0 = w0 / denom; w1 = w1 / denom; w2 = w2 / denom

    h1 = _gelu_exact(jax.lax.dot(xb, pw1_ref[...]) + pb1_ref[...])
    sc = jax.nn.sigmoid(jax.lax.dot(h1, pw2_ref[...]) + pb2_ref[0, 0])
    sc_t = jnp.transpose(sc)
    gate = jax.nn.sigmoid((sc * sc_t - thr_ref[0, 0]) * 10.0)

    out_ref[0] = ((adj * gate) * w0 + (adj * smask_ref[...]) * w1
                  + (adj * tmask_ref[...]) * w2)


def _full(shape):
    nd = len(shape)
    return pl.BlockSpec(shape, lambda b, _n=nd: (0,) * _n)


def kernel(x, proj_W, gate_W, gate_b, path_W1, path_b1, path_W2, path_b2, threshold):
    gate_b2 = gate_b.reshape(1, F)
    path_b1_2 = path_b1.reshape(1, D // 2)
    path_b2_2 = path_b2.reshape(1, 1)
    thr2 = threshold.reshape(1, 1)

    sums = pl.pallas_call(
        _sum_body,
        grid=(B,),
        in_specs=[pl.BlockSpec((1, N, D), lambda b: (b, 0, 0)), _full((H, D, HD))],
        out_specs=pl.BlockSpec((1, 1, 8), lambda b: (b, 0, 0)),
        out_shape=jax.ShapeDtypeStruct((B, 1, 8), jnp.float32),
        compiler_params=pltpu.CompilerParams(
            dimension_semantics=("parallel",)),
    )(x, proj_W)

    out = pl.pallas_call(
        _main_body,
        grid=(B,),
        in_specs=[
            pl.BlockSpec((1, N, D), lambda b: (b, 0, 0)),
            _full((H, D, HD)),
            _full((D, F)),
            _full((1, F)),
            _full((D, D // 2)),
            _full((1, D // 2)),
            _full((D // 2, 1)),
            _full((1, 1)),
            _full((1, 1)),
            _full((B, 1, 8)),
            _full((N, N)),
            _full((N, N)),
        ],
        out_specs=pl.BlockSpec((1, N, N), lambda b: (b, 0, 0)),
        out_shape=jax.ShapeDtypeStruct((B, N, N), jnp.float32),
        compiler_params=pltpu.CompilerParams(
            dimension_semantics=("parallel",)),
    )(x, proj_W, gate_W, gate_b2, path_W1, path_b1_2, path_W2, path_b2_2,
      thr2, sums, _SMASK, _TMASK)
    return out


# speedup vs baseline: 25.7552x; 1.0079x over previous
import numpy as np
import jax
import jax.numpy as jnp
from jax.experimental import pallas as pl
from jax.experimental.pallas import tpu as pltpu

B = 32; N_CH = 22; N_P = 20; N = N_CH * N_P; D = 128; H = 4; HD = D // H; F = 3
ALPHA = 0.15; TOP_P = 0.85; DIST_THRESH = 0.55
K = max(1, int(N * ALPHA))
T_BS = 22

_TCP_PAIRS = [('FP1','F7'),('F7','T3'),('T3','T5'),('T5','O1'),('FP2','F8'),('F8','T4'),('T4','T6'),('T6','O2'),('FP1','F3'),('F3','C3'),('C3','P3'),('P3','O1'),('FP2','F4'),('F4','C4'),('C4','P4'),('P4','O2'),('A1','T3'),('T3','C3'),('C3','CZ'),('CZ','C4'),('C4','T4'),('T4','A2')]
_E3D = {'FP1':(-0.31,0.95,0.0),'FP2':(0.31,0.95,0.0),'F7':(-0.81,0.59,0.0),'F3':(-0.55,0.67,0.5),'FZ':(0.0,0.71,0.71),'F4':(0.55,0.67,0.5),'F8':(0.81,0.59,0.0),'T3':(-1.0,0.0,0.0),'C3':(-0.57,0.0,0.82),'CZ':(0.0,0.0,1.0),'C4':(0.57,0.0,0.82),'T4':(1.0,0.0,0.0),'T5':(-0.81,-0.59,0.0),'P3':(-0.55,-0.67,0.5),'PZ':(0.0,-0.71,0.71),'P4':(0.55,-0.67,0.5),'T6':(0.81,-0.59,0.0),'O1':(-0.31,-0.95,0.0),'O2':(0.31,-0.95,0.0),'A1':(-1.05,0.0,-0.3),'A2':(1.05,0.0,-0.3)}


def _build_static_masks():
    ch_pos = np.array([(np.array(_E3D[a]) + np.array(_E3D[b])) / 2.0 for a, b in _TCP_PAIRS])
    dif = ch_pos[:, None, :] - ch_pos[None, :, :]
    ch_dist = np.sqrt((dif ** 2).sum(-1))
    ch_adj = (ch_dist < DIST_THRESH).astype(np.float32)
    np.fill_diagonal(ch_adj, 0.0)
    smask = np.zeros((N, N), np.float32)
    for t in range(N_P):
        idx = np.arange(N_CH) * N_P + t
        smask[np.ix_(idx, idx)] = ch_adj
    tmask = np.zeros((N, N), np.float32)
    for ch in range(N_CH):
        for p in range(N_P):
            node = ch * N_P + p
            if p > 0:
                tmask[node, node - 1] = 1.0
            if p < N_P - 1:
                tmask[node, node + 1] = 1.0
    return jnp.asarray(smask), jnp.asarray(tmask)


_SMASK, _TMASK = _build_static_masks()


def _erf(v):
    s = jnp.where(v >= 0.0, 1.0, -1.0)
    a = jnp.abs(v)
    t = 1.0 / (1.0 + 0.3275911 * a)
    poly = t * (0.254829592 + t * (-0.284496736 + t * (1.421413741
               + t * (-1.453152027 + t * 1.061405429))))
    return s * (1.0 - poly * jnp.exp(-a * a))


def _gelu_exact(v):
    return 0.5 * v * (1.0 + _erf(v * np.float32(1.0 / np.sqrt(2.0))))


def _head_d2(xb, pw):
    z = jax.lax.dot(xb, pw)
    sq = jnp.sum(z * z, axis=-1, keepdims=True)
    g = jax.lax.dot_general(z, z, (((1,), (1,)), ((), ())))
    return z, sq, g


def _sum_body(x_ref, pw_ref, sums_ref):
    xb = x_ref[0]
    parts = []
    for h in range(H):
        _, sq, g = _head_d2(xb, pw_ref[h])
        sq_t = jnp.transpose(sq)
        d2 = sq + sq_t - 2.0 * g
        dist = jnp.sqrt(jnp.maximum(d2, 1e-12))
        parts.append(jnp.sum(dist).reshape(1, 1))
    row = jnp.concatenate(parts + [jnp.zeros((1, 8 - H), jnp.float32)], axis=1)
    sums_ref[0] = row


def _main_body(x_ref, pw_ref, gw_ref, gb_ref, pw1_ref, pb1_ref, pw2_ref,
               pb2_ref, thr_ref, sums_ref, smask_ref, tmask_ref, out_ref):
    xb = x_ref[0]
    head_sums = jnp.sum(sums_ref[...], axis=0)

    d2s = []
    for h in range(H):
        _, sq, g = _head_d2(xb, pw_ref[h])
        sq_t = jnp.transpose(sq)
        d2s.append(sq + sq_t - 2.0 * g)
    d2a = jnp.concatenate(d2s, axis=0)

    ones_col = jnp.ones((N, 1), jnp.float32)
    lo = jnp.zeros((H * N, 1), jnp.float32)
    hi = jnp.max(d2a, axis=1, keepdims=True)
    for _ in range(T_BS):
        mid = 0.5 * (lo + hi)
        cmpf = jnp.where(d2a <= mid, 1.0, 0.0)
        cnt = jax.lax.dot(cmpf, ones_col)
        pred = cnt >= K
        lo = jnp.where(pred, lo, mid)
        hi = jnp.where(pred, mid, hi)

    adj = None
    for h in range(H):
        d2 = d2s[h]
        t = jax.lax.slice(hi, (h * N, 0), ((h + 1) * N, 1))
        t_t = jnp.transpose(t)
        maskf = jnp.where(d2 <= jnp.maximum(t, t_t), 1.0, 0.0)
        mean_h = head_sums[0, h] / float(B * N * N)
        neg_inv_c = -1.0 / (2.0 * mean_h * mean_h + 1e-08)
        sim = jnp.exp(jnp.maximum(d2, 1e-12) * neg_inv_c)
        contrib = sim * maskf
        adj = contrib if adj is None else adj + contrib
    adj = adj / float(H)

    logits = jax.lax.dot(xb, gw_ref[...]) + gb_ref[...]
    probs = jax.nn.softmax(logits, axis=-1)
    p0 = probs[:, 0:1]; p1 = probs[:, 1:2]; p2 = probs[:, 2:3]
    z0 = jnp.zeros_like(p0)
    bef0 = jnp.where(p1 > p0, p1, z0) + jnp.where(p2 > p0, p2, z0)
    bef1 = jnp.where(p0 >= p1, p0, z0) + jnp.where(p2 > p1, p2, z0)
    bef2 = jnp.where(p0 >= p2, p0, z0) + jnp.where(p1 >= p2, p1, z0)
    w0 = jnp.where(bef0 < TOP_P, p0, z0)
    w1 = jnp.where(bef1 < TOP_P, p1, z0)
    w2 = jnp.where(bef2 < TOP_P, p2, z0)
    denom = w0 + w1 + w2 + 1e-08
    w0 = w0 / denom; w1 = w1 / denom; w2 = w2 / denom

    h1 = _gelu_exact(jax.lax.dot(xb, pw1_ref[...]) + pb1_ref[...])
    sc = jax.nn.sigmoid(jax.lax.dot(h1, pw2_ref[...]) + pb2_ref[0, 0])
    sc_t = jnp.transpose(sc)
    gate = jax.nn.sigmoid((sc * sc_t - thr_ref[0, 0]) * 10.0)

    out_ref[0] = adj * (gate * w0 + smask_ref[...] * w1 + tmask_ref[...] * w2)


def _full(shape):
    nd = len(shape)
    return pl.BlockSpec(shape, lambda b, _n=nd: (0,) * _n)


def kernel(x, proj_W, gate_W, gate_b, path_W1, path_b1, path_W2, path_b2, threshold):
    gate_b2 = gate_b.reshape(1, F)
    path_b1_2 = path_b1.reshape(1, D // 2)
    path_b2_2 = path_b2.reshape(1, 1)
    thr2 = threshold.reshape(1, 1)

    sums = pl.pallas_call(
        _sum_body,
        grid=(B,),
        in_specs=[pl.BlockSpec((1, N, D), lambda b: (b, 0, 0)), _full((H, D, HD))],
        out_specs=pl.BlockSpec((1, 1, 8), lambda b: (b, 0, 0)),
        out_shape=jax.ShapeDtypeStruct((B, 1, 8), jnp.float32),
        compiler_params=pltpu.CompilerParams(
            dimension_semantics=("parallel",)),
    )(x, proj_W)

    out = pl.pallas_call(
        _main_body,
        grid=(B,),
        in_specs=[
            pl.BlockSpec((1, N, D), lambda b: (b, 0, 0)),
            _full((H, D, HD)),
            _full((D, F)),
            _full((1, F)),
            _full((D, D // 2)),
            _full((1, D // 2)),
            _full((D // 2, 1)),
            _full((1, 1)),
            _full((1, 1)),
            _full((B, 1, 8)),
            _full((N, N)),
            _full((N, N)),
        ],
        out_specs=pl.BlockSpec((1, N, N), lambda b: (b, 0, 0)),
        out_shape=jax.ShapeDtypeStruct((B, N, N), jnp.float32),
        compiler_params=pltpu.CompilerParams(
            dimension_semantics=("parallel",)),
    )(x, proj_W, gate_W, gate_b2, path_W1, path_b1_2, path_W2, path_b2_2,
      thr2, sums, _SMASK, _TMASK)
    return out
